# baseline (speedup 1.0000x reference)
import functools

import jax
import jax.numpy as jnp
from jax import lax
from jax.experimental import pallas as pl
from jax.experimental.pallas import tpu as pltpu
from jax.experimental.pallas import tpu_sc as plsc


def _make_sc_gather_sum(N, E, D, S, base_e, ch):
    info = plsc.get_sparse_core_info()
    NC, NS = info.num_cores, info.num_subcores
    NW = NC * NS
    epw = S // NW
    nchunk = epw // ch
    nv = D // 16
    mesh = plsc.VectorSubcoreMesh(core_axis_name="c", subcore_axis_name="s")

    @functools.partial(
        pl.kernel,
        out_type=jax.ShapeDtypeStruct((S, D), jnp.float32),
        mesh=mesh,
        scratch_types=[pltpu.VMEM((epw,), jnp.int32),
                       pltpu.VMEM((epw,), jnp.int32),
                       pltpu.VMEM((ch, D), jnp.float32),
                       pltpu.VMEM((ch, D), jnp.float32),
                       pltpu.VMEM((ch, D), jnp.float32),
                       pltpu.VMEM((ch, D), jnp.float32),
                       pltpu.VMEM_SHARED((N, D), jnp.float32),
                       pltpu.SemaphoreType.DMA,
                       pltpu.SemaphoreType.DMA,
                       pltpu.SemaphoreType.DMA,
                       pltpu.SemaphoreType.DMA],
    )
    def gather_k(pa_hbm, pb_hbm, src_hbm, dst_hbm, hsum_hbm,
                 idx_all_a, idx_all_b,
                 rows_a0, rows_b0, rows_a1, rows_b1,
                 spa,
                 sem_a0, sem_b0, sem_a1, sem_b1):
        cid = lax.axis_index("c")
        sid = lax.axis_index("s")
        wid = sid * NC + cid
        lbase = wid * epw
        @pl.when(sid == 0)
        def _():
            pltpu.sync_copy(pa_hbm, spa)

        pltpu.sync_copy(dst_hbm.at[pl.ds(base_e + lbase, epw)], idx_all_a)
        pltpu.sync_copy(src_hbm.at[pl.ds(base_e + lbase, epw)], idx_all_b)
        plsc.subcore_barrier()
        banks = ((rows_a0, rows_b0, sem_a0, sem_b0),
                 (rows_a1, rows_b1, sem_a1, sem_b1))

        def start(bank, ci):
            rows_a, rows_b, sem_a, sem_b = banks[bank]
            loff = ci * ch
            pltpu.async_copy(spa.at[idx_all_a.at[pl.ds(loff, ch)]],
                             rows_a, sem_a)
            pltpu.async_copy(pb_hbm.at[idx_all_b.at[pl.ds(loff, ch)]],
                             rows_b, sem_b)

        def finish(bank, ci):
            rows_a, rows_b, sem_a, sem_b = banks[bank]
            off = lbase + ci * ch
            loff = ci * ch
            pltpu.make_async_copy(spa.at[idx_all_a.at[pl.ds(loff, ch)]],
                                  rows_a, sem_a).wait()
            pltpu.make_async_copy(pb_hbm.at[idx_all_b.at[pl.ds(loff, ch)]],
                                  rows_b, sem_b).wait()

            def radd(r8, _):
                for u in range(8):
                    r = r8 * 8 + u
                    for c in range(nv):
                        plsc.addupdate(rows_a.at[r, pl.ds(c * 16, 16)],
                                       rows_b[r, pl.ds(c * 16, 16)])
                return 0

            lax.fori_loop(0, ch // 8, radd, 0)
            pltpu.sync_copy(rows_a, hsum_hbm.at[pl.ds(off, ch)])

        start(0, 0)

        def body(g, _):
            c0 = 2 * g
            c1 = 2 * g + 1

            @pl.when(c1 < nchunk)
            def _():
                start(1, c1)

            finish(0, c0)

            @pl.when(c1 + 1 < nchunk)
            def _():
                start(0, c1 + 1)

            @pl.when(c1 < nchunk)
            def _():
                finish(1, c1)

            return 0

        lax.fori_loop(0, (nchunk + 1) // 2, body, 0)

    return gather_k


def _make_sc_scatter(Np, E, D, ranges, ch):
    info = plsc.get_sparse_core_info()
    NC, NS = info.num_cores, info.num_subcores
    rpt = Np // NS
    mesh = plsc.VectorSubcoreMesh(core_axis_name="c", subcore_axis_name="s")

    @functools.partial(
        pl.kernel,
        out_type=jax.ShapeDtypeStruct((NC, Np, D), jnp.float32),
        mesh=mesh,
        scratch_types=[pltpu.VMEM((ch,), jnp.int32),
                       pltpu.VMEM((ch,), jnp.int32),
                       pltpu.VMEM((ch, D), jnp.float32),
                       pltpu.VMEM((ch, D), jnp.float32),
                       pltpu.VMEM_SHARED((Np, D), jnp.float32),
                       pltpu.SemaphoreType.DMA,
                       pltpu.SemaphoreType.DMA,
                       pltpu.SemaphoreType.DMA,
                       pltpu.SemaphoreType.DMA],
    )
    def scatter_k(*args):
        msg_hbms = args[:len(ranges)]
        (dst_hbm, zeros_hbm, out_hbm,
         idx0, idx1, rows0, rows1, acc_sh,
         sem_i0, sem_m0, sem_i1, sem_m1) = args[len(ranges):]
        cid = lax.axis_index("c")
        sid = lax.axis_index("s")
        pltpu.sync_copy(zeros_hbm.at[pl.ds(sid * rpt, rpt)],
                        acc_sh.at[pl.ds(sid * rpt, rpt)])
        plsc.subcore_barrier()

        banks = ((idx0, rows0, sem_i0, sem_m0),
                 (idx1, rows1, sem_i1, sem_m1))

        for msg_hbm, (S, base_e) in zip(msg_hbms, ranges):
            epc = S // NC
            ept = epc // NS
            nchunk = ept // ch
            lbase = cid * epc + sid * ept

            def start(bank, ci, msg_hbm=msg_hbm, lbase=lbase, base_e=base_e):
                idx_v, rows_v, sem_i, sem_m = banks[bank]
                off = lbase + ci * ch
                pltpu.async_copy(dst_hbm.at[pl.ds(base_e + off, ch)],
                                 idx_v, sem_i)
                pltpu.async_copy(msg_hbm.at[pl.ds(off, ch)], rows_v, sem_m)

            def finish(bank, ci, msg_hbm=msg_hbm, lbase=lbase, base_e=base_e):
                idx_v, rows_v, sem_i, sem_m = banks[bank]
                off = lbase + ci * ch
                pltpu.make_async_copy(dst_hbm.at[pl.ds(base_e + off, ch)],
                                      idx_v, sem_i).wait()
                pltpu.make_async_copy(msg_hbm.at[pl.ds(off, ch)],
                                      rows_v, sem_m).wait()
                pltpu.sync_copy(rows_v, acc_sh.at[idx_v], add=True)

            start(0, 0)

            def body(g, _, start=start, finish=finish, nchunk=nchunk):
                c0 = 2 * g
                c1 = 2 * g + 1

                @pl.when(c1 < nchunk)
                def _():
                    start(1, c1)

                finish(0, c0)

                @pl.when(c1 + 1 < nchunk)
                def _():
                    start(0, c1 + 1)

                @pl.when(c1 < nchunk)
                def _():
                    finish(1, c1)

                return 0

            lax.fori_loop(0, (nchunk + 1) // 2, body, 0)

        plsc.subcore_barrier()
        pltpu.sync_copy(acc_sh.at[pl.ds(sid * rpt, rpt)],
                        out_hbm.at[cid, pl.ds(sid * rpt, rpt)])

    return scatter_k


def _proj_body(x_ref, W0_ref, pa_ref, pb_ref):
    D = x_ref.shape[1]
    x = x_ref[...]
    pa_ref[...] = jnp.dot(x, W0_ref[0:D, :], preferred_element_type=jnp.float32)
    pb_ref[...] = jnp.dot(x, W0_ref[D:2 * D, :],
                          preferred_element_type=jnp.float32)


def _tc_proj(x, W0, block):
    N, D = x.shape
    grid = (N // block,)
    row_spec = pl.BlockSpec((block, D), lambda i: (i, 0))
    return pl.pallas_call(
        _proj_body,
        grid=grid,
        in_specs=[row_spec,
                  pl.BlockSpec((3 * D, D), lambda i: (0, 0))],
        out_specs=[row_spec, row_spec],
        out_shape=[jax.ShapeDtypeStruct((N, D), jnp.float32),
                   jax.ShapeDtypeStruct((N, D), jnp.float32)],
    )(x, W0)


def _edge_mlp_body(hs_ref, ea_ref, W0_ref, b0_ref, W1_ref, b1_ref,
                   W2_ref, b2_ref, g_ref, beta_ref, msg_ref, eout_ref):
    D = hs_ref.shape[1]
    ea = ea_ref[...]
    h = (hs_ref[...]
         + jnp.dot(ea, W0_ref[2 * D:3 * D, :], preferred_element_type=jnp.float32)
         + b0_ref[...])
    h = jnp.maximum(h, 0.0)
    h = jnp.maximum(jnp.dot(h, W1_ref[...], preferred_element_type=jnp.float32)
                    + b1_ref[...], 0.0)
    h = jnp.dot(h, W2_ref[...], preferred_element_type=jnp.float32) + b2_ref[...]
    mu = jnp.mean(h, axis=-1, keepdims=True)
    hc = h - mu
    var = jnp.mean(hc * hc, axis=-1, keepdims=True)
    hn = hc * lax.rsqrt(var + 1e-5)
    msg = hn * g_ref[...] + beta_ref[...]
    msg_ref[...] = msg
    eout_ref[...] = msg + ea


def _edge_mlp_alias_body(hs_ref, ea_ref, W0_ref, b0_ref, W1_ref,
                         b1_ref, W2_ref, b2_ref, g_ref, beta_ref, prev_ref,
                         msg_ref, eout_ref):
    del prev_ref
    _edge_mlp_body(hs_ref, ea_ref, W0_ref, b0_ref, W1_ref, b1_ref,
                   W2_ref, b2_ref, g_ref, beta_ref, msg_ref, eout_ref)


def _tc_edge_mlp_slice(k, blk_off, hs_k, ea, W0, b0, W1, b1, W2, b2, g, beta,
                       ea_out_prev, block):
    E, D = ea.shape
    S = hs_k.shape[0]
    nblk = S // block
    grid = (nblk,)
    loc = pl.BlockSpec((block, D), lambda i: (i, 0))
    glob = pl.BlockSpec((block, D), lambda i, o=blk_off: (o + i, 0))
    full = lambda shape: pl.BlockSpec(shape, lambda i: (0,) * len(shape))
    in_specs = [loc, glob,
                full((3 * D, D)), full((1, D)),
                full((D, D)), full((1, D)),
                full((D, D)), full((1, D)),
                full((1, D)), full((1, D))]
    args = [hs_k, ea, W0, b0.reshape(1, D), W1, b1.reshape(1, D),
            W2, b2.reshape(1, D), g.reshape(1, D), beta.reshape(1, D)]
    if k == 0:
        body = _edge_mlp_body
        aliases = {}
    else:
        body = _edge_mlp_alias_body
        in_specs.append(pl.BlockSpec(memory_space=pl.ANY))
        args.append(ea_out_prev)
        aliases = {10: 1}
    return pl.pallas_call(
        body,
        grid=grid,
        in_specs=in_specs,
        out_specs=[loc, glob],
        out_shape=[jax.ShapeDtypeStruct((S, D), jnp.float32),
                   jax.ShapeDtypeStruct((E, D), jnp.float32)],
        input_output_aliases=aliases,
    )(*args)


def _node_mlp_body(*refs):
    x_ref = refs[0]
    part_refs = refs[1:-9]
    W0_ref, b0_ref, W1_ref, b1_ref, W2_ref, b2_ref, g_ref, beta_ref = refs[-9:-1]
    out_ref = refs[-1]
    D = x_ref.shape[1]
    x = x_ref[...]
    aggr = part_refs[0][0]
    for r in part_refs[1:]:
        aggr = aggr + r[0]
    h = (jnp.dot(x, W0_ref[0:D, :], preferred_element_type=jnp.float32)
         + jnp.dot(aggr, W0_ref[D:2 * D, :], preferred_element_type=jnp.float32)
         + b0_ref[...])
    h = jnp.maximum(h, 0.0)
    h = jnp.maximum(jnp.dot(h, W1_ref[...], preferred_element_type=jnp.float32)
                    + b1_ref[...], 0.0)
    h = jnp.dot(h, W2_ref[...], preferred_element_type=jnp.float32) + b2_ref[...]
    mu = jnp.mean(h, axis=-1, keepdims=True)
    hc = h - mu
    var = jnp.mean(hc * hc, axis=-1, keepdims=True)
    hn = hc * lax.rsqrt(var + 1e-5)
    out_ref[...] = hn * g_ref[...] + beta_ref[...] + x


def _tc_node_mlp(x, partials, W0, b0, W1, b1, W2, b2, g, beta, block):
    N, D = x.shape
    grid = (N // block,)
    row_spec = pl.BlockSpec((block, D), lambda i: (i, 0))
    full = lambda shape: pl.BlockSpec(shape, lambda i: (0,) * len(shape))
    part_specs = []
    part_args = []
    for p in partials:
        NC = p.shape[0]
        for c in range(NC):
            part_specs.append(
                pl.BlockSpec((1, block, D), lambda i, c=c: (c, i, 0)))
            part_args.append(p)
    return pl.pallas_call(
        _node_mlp_body,
        grid=grid,
        in_specs=[row_spec] + part_specs +
                 [full((2 * D, D)), full((1, D)),
                  full((D, D)), full((1, D)),
                  full((D, D)), full((1, D)),
                  full((1, D)), full((1, D))],
        out_specs=row_spec,
        out_shape=jax.ShapeDtypeStruct((N, D), jnp.float32),
    )(x, *part_args, W0, b0.reshape(1, D), W1, b1.reshape(1, D),
      W2, b2.reshape(1, D), g.reshape(1, D), beta.reshape(1, D))


def kernel(x, edge_index, edge_attr, mW0, mb0, mW1, mb1, mW2, mb2, mg, mbeta,
           uW0, ub0, uW1, ub1, uW2, ub2, ug, ubeta):
    N, D = x.shape
    E = edge_attr.shape[0]
    src = edge_index[0]
    dst = edge_index[1]

    info = plsc.get_sparse_core_info()
    NS = info.num_subcores
    Np = ((N + 8 * NS - 1) // (8 * NS)) * (8 * NS)
    zeros = jnp.zeros((Np, D), jnp.float32)

    Pa, Pb = _tc_proj(x, mW0, block=1000)

    unit = E // 25
    sizes = [2 * unit, 6 * unit, 7 * unit, 7 * unit, 3 * unit]
    block = 1600
    ea_out = None
    msgs = []
    bases = []
    base = 0
    blk_off = 0
    for k, S in enumerate(sizes):
        hs_k = _make_sc_gather_sum(N, E, D, S, base, ch=80)(Pa, Pb, src, dst)
        msg_k, ea_out = _tc_edge_mlp_slice(
            k, blk_off, hs_k, edge_attr,
            mW0, mb0, mW1, mb1, mW2, mb2, mg, mbeta, ea_out, block=block)
        msgs.append(msg_k)
        bases.append(base)
        base += S
        blk_off += S // block

    groups = [(0, 1), (2, 3), (4,)]
    partials = []
    for grp in groups:
        ranges = [(sizes[k], bases[k]) for k in grp]
        partials.append(
            _make_sc_scatter(Np, E, D, ranges, ch=80)(
                *[msgs[k] for k in grp], dst, zeros))

    x_out = _tc_node_mlp(x, partials,
                         uW0, ub0, uW1, ub1, uW2, ub2, ug, ubeta,
                         block=1000)
    return (x_out, ea_out)


# speedup vs baseline: 1.0245x; 1.0245x over previous
import functools

import jax
import jax.numpy as jnp
from jax import lax
from jax.experimental import pallas as pl
from jax.experimental.pallas import tpu as pltpu
from jax.experimental.pallas import tpu_sc as plsc


def _make_sc_gather_sum(N, E, D, S, base_e, ch):
    info = plsc.get_sparse_core_info()
    NC, NS = info.num_cores, info.num_subcores
    NW = NC * NS
    epw = S // NW
    nchunk = epw // ch
    nv = D // 16
    mesh = plsc.VectorSubcoreMesh(core_axis_name="c", subcore_axis_name="s")

    @functools.partial(
        pl.kernel,
        out_type=jax.ShapeDtypeStruct((S, D), jnp.float32),
        mesh=mesh,
        scratch_types=[pltpu.VMEM((epw,), jnp.int32),
                       pltpu.VMEM((epw,), jnp.int32),
                       pltpu.VMEM((ch, D), jnp.float32),
                       pltpu.VMEM((ch, D), jnp.float32),
                       pltpu.VMEM((ch, D), jnp.float32),
                       pltpu.VMEM((ch, D), jnp.float32),
                       pltpu.VMEM_SHARED((N, D), jnp.float32),
                       pltpu.SemaphoreType.DMA,
                       pltpu.SemaphoreType.DMA,
                       pltpu.SemaphoreType.DMA,
                       pltpu.SemaphoreType.DMA],
    )
    def gather_k(pa_hbm, pb_hbm, src_hbm, dst_hbm, hsum_hbm,
                 idx_all_a, idx_all_b,
                 rows_a0, rows_b0, rows_a1, rows_b1,
                 spa,
                 sem_a0, sem_b0, sem_a1, sem_b1):
        cid = lax.axis_index("c")
        sid = lax.axis_index("s")
        wid = sid * NC + cid
        lbase = wid * epw
        @pl.when(sid == 0)
        def _():
            pltpu.sync_copy(pa_hbm, spa)

        pltpu.sync_copy(dst_hbm.at[pl.ds(base_e + lbase, epw)], idx_all_a)
        pltpu.sync_copy(src_hbm.at[pl.ds(base_e + lbase, epw)], idx_all_b)
        plsc.subcore_barrier()
        banks = ((rows_a0, rows_b0, sem_a0, sem_b0),
                 (rows_a1, rows_b1, sem_a1, sem_b1))

        def start(bank, ci):
            rows_a, rows_b, sem_a, sem_b = banks[bank]
            loff = ci * ch
            pltpu.async_copy(spa.at[idx_all_a.at[pl.ds(loff, ch)]],
                             rows_a, sem_a)
            pltpu.async_copy(pb_hbm.at[idx_all_b.at[pl.ds(loff, ch)]],
                             rows_b, sem_b)

        def finish(bank, ci):
            rows_a, rows_b, sem_a, sem_b = banks[bank]
            off = lbase + ci * ch
            loff = ci * ch
            pltpu.make_async_copy(spa.at[idx_all_a.at[pl.ds(loff, ch)]],
                                  rows_a, sem_a).wait()
            pltpu.make_async_copy(pb_hbm.at[idx_all_b.at[pl.ds(loff, ch)]],
                                  rows_b, sem_b).wait()

            def radd(r8, _):
                for u in range(8):
                    r = r8 * 8 + u
                    for c in range(nv):
                        plsc.addupdate(rows_a.at[r, pl.ds(c * 16, 16)],
                                       rows_b[r, pl.ds(c * 16, 16)])
                return 0

            lax.fori_loop(0, ch // 8, radd, 0)
            pltpu.sync_copy(rows_a, hsum_hbm.at[pl.ds(off, ch)])

        start(0, 0)

        def body(g, _):
            c0 = 2 * g
            c1 = 2 * g + 1

            @pl.when(c1 < nchunk)
            def _():
                start(1, c1)

            finish(0, c0)

            @pl.when(c1 + 1 < nchunk)
            def _():
                start(0, c1 + 1)

            @pl.when(c1 < nchunk)
            def _():
                finish(1, c1)

            return 0

        lax.fori_loop(0, (nchunk + 1) // 2, body, 0)

    return gather_k


def _make_sc_scatter(Np, E, D, ranges, ch):
    info = plsc.get_sparse_core_info()
    NC, NS = info.num_cores, info.num_subcores
    rpt = Np // NS
    mesh = plsc.VectorSubcoreMesh(core_axis_name="c", subcore_axis_name="s")

    @functools.partial(
        pl.kernel,
        out_type=jax.ShapeDtypeStruct((NC, Np, D), jnp.float32),
        mesh=mesh,
        scratch_types=[pltpu.VMEM((ch,), jnp.int32),
                       pltpu.VMEM((ch,), jnp.int32),
                       pltpu.VMEM((ch, D), jnp.float32),
                       pltpu.VMEM((ch, D), jnp.float32),
                       pltpu.VMEM_SHARED((Np, D), jnp.float32),
                       pltpu.SemaphoreType.DMA,
                       pltpu.SemaphoreType.DMA,
                       pltpu.SemaphoreType.DMA,
                       pltpu.SemaphoreType.DMA],
    )
    def scatter_k(*args):
        msg_hbms = args[:len(ranges)]
        (dst_hbm, zeros_hbm, out_hbm,
         idx0, idx1, rows0, rows1, acc_sh,
         sem_i0, sem_m0, sem_i1, sem_m1) = args[len(ranges):]
        cid = lax.axis_index("c")
        sid = lax.axis_index("s")
        pltpu.sync_copy(zeros_hbm.at[pl.ds(sid * rpt, rpt)],
                        acc_sh.at[pl.ds(sid * rpt, rpt)])
        plsc.subcore_barrier()

        banks = ((idx0, rows0, sem_i0, sem_m0),
                 (idx1, rows1, sem_i1, sem_m1))

        for msg_hbm, (S, base_e) in zip(msg_hbms, ranges):
            epc = S // NC
            ept = epc // NS
            nchunk = ept // ch
            lbase = cid * epc + sid * ept

            def start(bank, ci, msg_hbm=msg_hbm, lbase=lbase, base_e=base_e):
                idx_v, rows_v, sem_i, sem_m = banks[bank]
                off = lbase + ci * ch
                pltpu.async_copy(dst_hbm.at[pl.ds(base_e + off, ch)],
                                 idx_v, sem_i)
                pltpu.async_copy(msg_hbm.at[pl.ds(off, ch)], rows_v, sem_m)

            def finish(bank, ci, msg_hbm=msg_hbm, lbase=lbase, base_e=base_e):
                idx_v, rows_v, sem_i, sem_m = banks[bank]
                off = lbase + ci * ch
                pltpu.make_async_copy(dst_hbm.at[pl.ds(base_e + off, ch)],
                                      idx_v, sem_i).wait()
                pltpu.make_async_copy(msg_hbm.at[pl.ds(off, ch)],
                                      rows_v, sem_m).wait()
                pltpu.sync_copy(rows_v, acc_sh.at[idx_v], add=True)

            start(0, 0)

            def body(g, _, start=start, finish=finish, nchunk=nchunk):
                c0 = 2 * g
                c1 = 2 * g + 1

                @pl.when(c1 < nchunk)
                def _():
                    start(1, c1)

                finish(0, c0)

                @pl.when(c1 + 1 < nchunk)
                def _():
                    start(0, c1 + 1)

                @pl.when(c1 < nchunk)
                def _():
                    finish(1, c1)

                return 0

            lax.fori_loop(0, (nchunk + 1) // 2, body, 0)

        plsc.subcore_barrier()
        pltpu.sync_copy(acc_sh.at[pl.ds(sid * rpt, rpt)],
                        out_hbm.at[cid, pl.ds(sid * rpt, rpt)])

    return scatter_k


def _proj_body(x_ref, W0_ref, pa_ref, pb_ref):
    D = x_ref.shape[1]
    x = x_ref[...]
    pa_ref[...] = jnp.dot(x, W0_ref[0:D, :], preferred_element_type=jnp.float32)
    pb_ref[...] = jnp.dot(x, W0_ref[D:2 * D, :],
                          preferred_element_type=jnp.float32)


def _tc_proj(x, W0, block):
    N, D = x.shape
    grid = (N // block,)
    row_spec = pl.BlockSpec((block, D), lambda i: (i, 0))
    return pl.pallas_call(
        _proj_body,
        grid=grid,
        in_specs=[row_spec,
                  pl.BlockSpec((3 * D, D), lambda i: (0, 0))],
        out_specs=[row_spec, row_spec],
        out_shape=[jax.ShapeDtypeStruct((N, D), jnp.float32),
                   jax.ShapeDtypeStruct((N, D), jnp.float32)],
    )(x, W0)


def _edge_mlp_body(hs_ref, ea_ref, W0_ref, b0_ref, W1_ref, b1_ref,
                   W2_ref, b2_ref, g_ref, beta_ref, msg_ref, eout_ref):
    D = hs_ref.shape[1]
    ea = ea_ref[...]
    h = (hs_ref[...]
         + jnp.dot(ea, W0_ref[2 * D:3 * D, :], preferred_element_type=jnp.float32)
         + b0_ref[...])
    h = jnp.maximum(h, 0.0)
    h = jnp.maximum(jnp.dot(h, W1_ref[...], preferred_element_type=jnp.float32)
                    + b1_ref[...], 0.0)
    h = jnp.dot(h, W2_ref[...], preferred_element_type=jnp.float32) + b2_ref[...]
    mu = jnp.mean(h, axis=-1, keepdims=True)
    hc = h - mu
    var = jnp.mean(hc * hc, axis=-1, keepdims=True)
    hn = hc * lax.rsqrt(var + 1e-5)
    msg = hn * g_ref[...] + beta_ref[...]
    msg_ref[...] = msg
    eout_ref[...] = msg + ea


def _edge_mlp_alias_body(hs_ref, ea_ref, W0_ref, b0_ref, W1_ref,
                         b1_ref, W2_ref, b2_ref, g_ref, beta_ref, prev_ref,
                         msg_ref, eout_ref):
    del prev_ref
    _edge_mlp_body(hs_ref, ea_ref, W0_ref, b0_ref, W1_ref, b1_ref,
                   W2_ref, b2_ref, g_ref, beta_ref, msg_ref, eout_ref)


def _tc_edge_mlp_slice(k, blk_off, hs_k, ea, W0, b0, W1, b1, W2, b2, g, beta,
                       ea_out_prev, block):
    E, D = ea.shape
    S = hs_k.shape[0]
    nblk = S // block
    grid = (nblk,)
    loc = pl.BlockSpec((block, D), lambda i: (i, 0))
    glob = pl.BlockSpec((block, D), lambda i, o=blk_off: (o + i, 0))
    full = lambda shape: pl.BlockSpec(shape, lambda i: (0,) * len(shape))
    in_specs = [loc, glob,
                full((3 * D, D)), full((1, D)),
                full((D, D)), full((1, D)),
                full((D, D)), full((1, D)),
                full((1, D)), full((1, D))]
    args = [hs_k, ea, W0, b0.reshape(1, D), W1, b1.reshape(1, D),
            W2, b2.reshape(1, D), g.reshape(1, D), beta.reshape(1, D)]
    if k == 0:
        body = _edge_mlp_body
        aliases = {}
    else:
        body = _edge_mlp_alias_body
        in_specs.append(pl.BlockSpec(memory_space=pl.ANY))
        args.append(ea_out_prev)
        aliases = {10: 1}
    return pl.pallas_call(
        body,
        grid=grid,
        in_specs=in_specs,
        out_specs=[loc, glob],
        out_shape=[jax.ShapeDtypeStruct((S, D), jnp.float32),
                   jax.ShapeDtypeStruct((E, D), jnp.float32)],
        input_output_aliases=aliases,
    )(*args)


def _node_mlp_body(*refs):
    x_ref = refs[0]
    part_refs = refs[1:-9]
    W0_ref, b0_ref, W1_ref, b1_ref, W2_ref, b2_ref, g_ref, beta_ref = refs[-9:-1]
    out_ref = refs[-1]
    D = x_ref.shape[1]
    x = x_ref[...]
    aggr = part_refs[0][0]
    for r in part_refs[1:]:
        aggr = aggr + r[0]
    h = (jnp.dot(x, W0_ref[0:D, :], preferred_element_type=jnp.float32)
         + jnp.dot(aggr, W0_ref[D:2 * D, :], preferred_element_type=jnp.float32)
         + b0_ref[...])
    h = jnp.maximum(h, 0.0)
    h = jnp.maximum(jnp.dot(h, W1_ref[...], preferred_element_type=jnp.float32)
                    + b1_ref[...], 0.0)
    h = jnp.dot(h, W2_ref[...], preferred_element_type=jnp.float32) + b2_ref[...]
    mu = jnp.mean(h, axis=-1, keepdims=True)
    hc = h - mu
    var = jnp.mean(hc * hc, axis=-1, keepdims=True)
    hn = hc * lax.rsqrt(var + 1e-5)
    out_ref[...] = hn * g_ref[...] + beta_ref[...] + x


def _tc_node_mlp(x, partials, W0, b0, W1, b1, W2, b2, g, beta, block):
    N, D = x.shape
    grid = (N // block,)
    row_spec = pl.BlockSpec((block, D), lambda i: (i, 0))
    full = lambda shape: pl.BlockSpec(shape, lambda i: (0,) * len(shape))
    part_specs = []
    part_args = []
    for p in partials:
        NC = p.shape[0]
        for c in range(NC):
            part_specs.append(
                pl.BlockSpec((1, block, D), lambda i, c=c: (c, i, 0)))
            part_args.append(p)
    return pl.pallas_call(
        _node_mlp_body,
        grid=grid,
        in_specs=[row_spec] + part_specs +
                 [full((2 * D, D)), full((1, D)),
                  full((D, D)), full((1, D)),
                  full((D, D)), full((1, D)),
                  full((1, D)), full((1, D))],
        out_specs=row_spec,
        out_shape=jax.ShapeDtypeStruct((N, D), jnp.float32),
    )(x, *part_args, W0, b0.reshape(1, D), W1, b1.reshape(1, D),
      W2, b2.reshape(1, D), g.reshape(1, D), beta.reshape(1, D))


def kernel(x, edge_index, edge_attr, mW0, mb0, mW1, mb1, mW2, mb2, mg, mbeta,
           uW0, ub0, uW1, ub1, uW2, ub2, ug, ubeta):
    N, D = x.shape
    E = edge_attr.shape[0]
    src = edge_index[0]
    dst = edge_index[1]

    info = plsc.get_sparse_core_info()
    NS = info.num_subcores
    Np = ((N + 8 * NS - 1) // (8 * NS)) * (8 * NS)
    zeros = jnp.zeros((Np, D), jnp.float32)

    Pa, Pb = _tc_proj(x, mW0, block=1000)

    unit = E // 25
    sizes = [2 * unit, 6 * unit, 7 * unit, 7 * unit, 3 * unit]
    block = 1600
    ea_out = None
    msgs = []
    bases = []
    base = 0
    blk_off = 0
    for k, S in enumerate(sizes):
        hs_k = _make_sc_gather_sum(N, E, D, S, base, ch=80)(Pa, Pb, src, dst)
        msg_k, ea_out = _tc_edge_mlp_slice(
            k, blk_off, hs_k, edge_attr,
            mW0, mb0, mW1, mb1, mW2, mb2, mg, mbeta, ea_out, block=block)
        msgs.append(msg_k)
        bases.append(base)
        base += S
        blk_off += S // block

    groups = [(0, 1), (2,), (3,), (4,)]
    partials = []
    for grp in groups:
        ranges = [(sizes[k], bases[k]) for k in grp]
        partials.append(
            _make_sc_scatter(Np, E, D, ranges, ch=80)(
                *[msgs[k] for k in grp], dst, zeros))

    x_out = _tc_node_mlp(x, partials,
                         uW0, ub0, uW1, ub1, uW2, ub2, ug, ubeta,
                         block=1000)
    return (x_out, ea_out)
